# Initial kernel scaffold; baseline (speedup 1.0000x reference)
import jax
import jax.numpy as jnp
from jax.experimental import pallas as pl


def kernel(input_ids):
    raise NotImplementedError("write your pallas kernel here")


# speedup vs baseline: 1.4766x; 1.4766x over previous
import jax
import jax.numpy as jnp
from jax.experimental import pallas as pl

_VOCAB = 1000
_HI = 5.0
_LO = -5.0
_BR = 256


def _onehot_block(ids_ref, out_ref):
    ids = ids_ref[...]
    col = jax.lax.broadcasted_iota(jnp.int32, (_BR, _VOCAB), 1)
    out_ref[...] = jnp.where(col == ids, _HI, _LO)


def kernel(input_ids):
    Bx, Tx = input_ids.shape
    rows = Bx * Tx
    ids = (input_ids.astype(jnp.int32) % _VOCAB).reshape(rows, 1)
    out = pl.pallas_call(
        _onehot_block,
        grid=(rows // _BR,),
        in_specs=[pl.BlockSpec((_BR, 1), lambda i: (i, 0))],
        out_specs=pl.BlockSpec((_BR, _VOCAB), lambda i: (i, 0)),
        out_shape=jax.ShapeDtypeStruct((rows, _VOCAB), jnp.float32),
    )(ids)
    return out.reshape(Bx, Tx, _VOCAB)
